# Initial kernel scaffold; baseline (speedup 1.0000x reference)
import jax
import jax.numpy as jnp
from jax.experimental import pallas as pl


def kernel(x, W, b_lin, bias, k):
    raise NotImplementedError("write your pallas kernel here")


# speedup vs baseline: 40.8050x; 40.8050x over previous
import jax
import jax.numpy as jnp
from jax.experimental import pallas as pl
from jax.experimental.pallas import tpu as pltpu

_D_IN = 2048
_PKM = 1000
_PAD = 1024
_K = 32
_N_TOK = 8192
_BLK = 256
_NEG_PAD = -1e30
_DEAD = -3e38


def _pkm_body(x_ref, w_ref, b_ref, ow_ref, oi_ref):
    B = x_ref.shape[0]
    h = jnp.dot(x_ref[...], w_ref[...], preferred_element_type=jnp.float32)
    h = h + b_ref[...]
    col = jax.lax.broadcasted_iota(jnp.int32, (B, _PAD), 1)
    col_k = jax.lax.broadcasted_iota(jnp.int32, (B, _K), 1)
    big = jnp.int32(1 << 30)

    def topk_half(v):
        def step(j, carry):
            v, wa, pa = carry
            m = jnp.max(v, axis=1, keepdims=True)
            pos = jnp.min(jnp.where(v == m, col, big), axis=1, keepdims=True)
            sel = col == pos
            v = jnp.where(sel, _DEAD, v)
            wa = jnp.where(col_k == j, m, wa)
            pa = jnp.where(col_k == j, pos, pa)
            return v, wa, pa

        init = (v, jnp.zeros((B, _K), jnp.float32), jnp.zeros((B, _K), jnp.int32))
        _, wa, pa = jax.lax.fori_loop(0, _K, step, init)
        return wa, pa

    w1, p1 = topk_half(h[:, :_PAD])
    w2, p2 = topk_half(h[:, _PAD:])

    w1e = jnp.concatenate(
        [jnp.broadcast_to(w1[:, a:a + 1], (B, _K)) for a in range(_K)], axis=1)
    i1e = jnp.concatenate(
        [jnp.broadcast_to(p1[:, a:a + 1], (B, _K)) for a in range(_K)], axis=1)
    w2e = jnp.concatenate([w2] * _K, axis=1)
    i2e = jnp.concatenate([p2] * _K, axis=1)

    cand = jnp.maximum(w1e + w2e, 0.0)
    cand_i = i1e * _PKM + i2e

    def step2(j, carry):
        v, wa, ia = carry
        m = jnp.max(v, axis=1, keepdims=True)
        pos = jnp.min(jnp.where(v == m, col, big), axis=1, keepdims=True)
        sel = col == pos
        idx = jnp.sum(jnp.where(sel, cand_i, 0), axis=1, keepdims=True)
        v = jnp.where(sel, _DEAD, v)
        wa = jnp.where(col_k == j, m, wa)
        ia = jnp.where(col_k == j, idx, ia)
        return v, wa, ia

    init2 = (cand, jnp.zeros((B, _K), jnp.float32), jnp.zeros((B, _K), jnp.int32))
    _, wf, idxf = jax.lax.fori_loop(0, _K, step2, init2)
    ow_ref[...] = wf
    oi_ref[...] = idxf


def _run_pallas(x, wpt, bp):
    grid = (_N_TOK // _BLK,)
    return pl.pallas_call(
        _pkm_body,
        grid=grid,
        in_specs=[
            pl.BlockSpec((_BLK, _D_IN), lambda i: (i, 0)),
            pl.BlockSpec((_D_IN, 2 * _PAD), lambda i: (0, 0)),
            pl.BlockSpec((1, 2 * _PAD), lambda i: (0, 0)),
        ],
        out_specs=[
            pl.BlockSpec((_BLK, _K), lambda i: (i, 0)),
            pl.BlockSpec((_BLK, _K), lambda i: (i, 0)),
        ],
        out_shape=[
            jax.ShapeDtypeStruct((_N_TOK, _K), jnp.float32),
            jax.ShapeDtypeStruct((_N_TOK, _K), jnp.int32),
        ],
        compiler_params=pltpu.CompilerParams(
            dimension_semantics=("parallel",),
        ),
    )(x, wpt, bp)


def kernel(x, W, b_lin, bias, k):
    del bias
    npad = _PAD - _PKM
    zrows = jnp.zeros((npad, _D_IN), W.dtype)
    wpt = jnp.concatenate([W[:_PKM], zrows, W[_PKM:], zrows], axis=0).T
    negs = jnp.full((npad,), _NEG_PAD, jnp.float32)
    bp = jnp.concatenate(
        [b_lin[:_PKM], negs, b_lin[_PKM:], negs]).reshape(1, 2 * _PAD)
    w, i = _run_pallas(x, wpt, bp)
    keep = jnp.asarray(k) == _K
    w = jnp.where(keep, w, jnp.zeros_like(w))
    i = jnp.where(keep, i, jnp.zeros_like(i))
    return w, i


# speedup vs baseline: 45.8883x; 1.1246x over previous
import jax
import jax.numpy as jnp
from jax.experimental import pallas as pl
from jax.experimental.pallas import tpu as pltpu

_D_IN = 2048
_PKM = 1000
_PAD = 1024
_K = 32
_N_TOK = 8192
_BLK = 256
_NEG_PAD = -1e30
_DEAD = -3e38


_AB = [(a, b) for a in range(_K) for b in range(_K) if (a + 1) * (b + 1) <= _K]
_NCAND = 128


def _pkm_body(x_ref, w_ref, b_ref, ai_ref, bi_ref, ci_ref, pad_ref,
              ow_ref, oi_ref):
    B = x_ref.shape[0]
    h = jnp.dot(x_ref[...], w_ref[...], preferred_element_type=jnp.float32)
    h = h + b_ref[...]
    col = jax.lax.broadcasted_iota(jnp.int32, (B, _PAD), 1)
    col_k = jax.lax.broadcasted_iota(jnp.int32, (B, _K), 1)
    big = jnp.int32(1 << 30)

    def topk_half(v):
        def step(j, carry):
            v, wa, pa = carry
            m = jnp.max(v, axis=1, keepdims=True)
            pos = jnp.min(jnp.where(v == m, col, big), axis=1, keepdims=True)
            sel = col == pos
            v = jnp.where(sel, _DEAD, v)
            wa = jnp.where(col_k == j, m, wa)
            pa = jnp.where(col_k == j, pos, pa)
            return v, wa, pa

        init = (v, jnp.zeros((B, _K), jnp.float32), jnp.zeros((B, _K), jnp.int32))
        _, wa, pa = jax.lax.fori_loop(0, _K, step, init)
        return wa, pa

    w1, p1 = topk_half(h[:, :_PAD])
    w2, p2 = topk_half(h[:, _PAD:])

    ai = ai_ref[...]
    bi = bi_ref[...]
    zf = jnp.zeros((B, _NCAND), jnp.float32)
    zi = jnp.zeros((B, _NCAND), jnp.int32)
    w1e, i1e, w2e, i2e = zf, zi, zf, zi
    for a in range(_K):
        ma = ai == a
        mb = bi == a
        w1e = w1e + jnp.where(ma, w1[:, a:a + 1], 0.0)
        i1e = i1e + jnp.where(ma, p1[:, a:a + 1], 0)
        w2e = w2e + jnp.where(mb, w2[:, a:a + 1], 0.0)
        i2e = i2e + jnp.where(mb, p2[:, a:a + 1], 0)

    cand = jnp.maximum(w1e + w2e, 0.0) + pad_ref[...]
    cand_i = i1e * _PKM + i2e
    cidb = jnp.broadcast_to(ci_ref[...], (B, _NCAND))

    def step2(j, carry):
        v, wa, ia = carry
        m = jnp.max(v, axis=1, keepdims=True)
        pos = jnp.min(jnp.where(v == m, cidb, big), axis=1, keepdims=True)
        sel = cidb == pos
        idx = jnp.sum(jnp.where(sel, cand_i, 0), axis=1, keepdims=True)
        v = jnp.where(sel, _DEAD, v)
        wa = jnp.where(col_k == j, m, wa)
        ia = jnp.where(col_k == j, idx, ia)
        return v, wa, ia

    init2 = (cand, jnp.zeros((B, _K), jnp.float32), jnp.zeros((B, _K), jnp.int32))
    _, wf, idxf = jax.lax.fori_loop(0, _K, step2, init2)
    ow_ref[...] = wf
    oi_ref[...] = idxf


def _cand_tables():
    import numpy as np
    ai = np.full((1, _NCAND), _K + 7, np.int32)
    bi = np.full((1, _NCAND), _K + 7, np.int32)
    ci = np.full((1, _NCAND), 1 << 30, np.int32)
    padv = np.full((1, _NCAND), _DEAD, np.float32)
    for j, (a, b) in enumerate(_AB):
        ai[0, j] = a
        bi[0, j] = b
        ci[0, j] = a * _K + b
        padv[0, j] = 0.0
    return (jnp.asarray(ai), jnp.asarray(bi), jnp.asarray(ci),
            jnp.asarray(padv))


def _run_pallas(x, wpt, bp):
    ai, bi, ci, padv = _cand_tables()
    grid = (_N_TOK // _BLK,)
    fixed = lambda i: (0, 0)
    return pl.pallas_call(
        _pkm_body,
        grid=grid,
        in_specs=[
            pl.BlockSpec((_BLK, _D_IN), lambda i: (i, 0)),
            pl.BlockSpec((_D_IN, 2 * _PAD), fixed),
            pl.BlockSpec((1, 2 * _PAD), fixed),
            pl.BlockSpec((1, _NCAND), fixed),
            pl.BlockSpec((1, _NCAND), fixed),
            pl.BlockSpec((1, _NCAND), fixed),
            pl.BlockSpec((1, _NCAND), fixed),
        ],
        out_specs=[
            pl.BlockSpec((_BLK, _K), lambda i: (i, 0)),
            pl.BlockSpec((_BLK, _K), lambda i: (i, 0)),
        ],
        out_shape=[
            jax.ShapeDtypeStruct((_N_TOK, _K), jnp.float32),
            jax.ShapeDtypeStruct((_N_TOK, _K), jnp.int32),
        ],
        compiler_params=pltpu.CompilerParams(
            dimension_semantics=("parallel",),
        ),
    )(x, wpt, bp, ai, bi, ci, padv)


def kernel(x, W, b_lin, bias, k):
    del bias
    npad = _PAD - _PKM
    zrows = jnp.zeros((npad, _D_IN), W.dtype)
    wpt = jnp.concatenate([W[:_PKM], zrows, W[_PKM:], zrows], axis=0).T
    negs = jnp.full((npad,), _NEG_PAD, jnp.float32)
    bp = jnp.concatenate(
        [b_lin[:_PKM], negs, b_lin[_PKM:], negs]).reshape(1, 2 * _PAD)
    w, i = _run_pallas(x, wpt, bp)
    keep = jnp.asarray(k) == _K
    w = jnp.where(keep, w, jnp.zeros_like(w))
    i = jnp.where(keep, i, jnp.zeros_like(i))
    return w, i


# speedup vs baseline: 108.3686x; 2.3616x over previous
import functools

import jax
import jax.numpy as jnp
from jax import lax
from jax.experimental import pallas as pl
from jax.experimental.pallas import tpu as pltpu
from jax.experimental.pallas import tpu_sc as plsc

_D_IN = 2048
_PKM = 1000
_PAD = 1024
_K = 32
_N_TOK = 8192
_BLK = 256
_NEG_PAD = -1e30
_DEAD = -3e38

_NC = 2
_NS = 16
_L = 16
_NW = _NC * _NS
_RPW = _N_TOK // _NW
_RB = 16
_NBATCH = _RPW // _RB

_AB = [(a, b) for a in range(_K) for b in range(_K) if (a + 1) * (b + 1) <= _K]
_NCAND = 128


def _mm_body(x_ref, w_ref, b_ref, h_ref):
    h = jnp.dot(x_ref[...], w_ref[...], preferred_element_type=jnp.float32)
    h_ref[...] = h + b_ref[...]


def _matmul(x, wpt, bp):
    fixed = lambda i: (0, 0)
    return pl.pallas_call(
        _mm_body,
        grid=(_N_TOK // _BLK,),
        in_specs=[
            pl.BlockSpec((_BLK, _D_IN), lambda i: (i, 0)),
            pl.BlockSpec((_D_IN, 2 * _PAD), fixed),
            pl.BlockSpec((1, 2 * _PAD), fixed),
        ],
        out_specs=pl.BlockSpec((_BLK, 2 * _PAD), lambda i: (i, 0)),
        out_shape=jax.ShapeDtypeStruct((_N_TOK, 2 * _PAD), jnp.float32),
        compiler_params=pltpu.CompilerParams(
            dimension_semantics=("parallel",),
        ),
    )(x, wpt, bp)


def _sortkv(keys, vals):
    return plsc.sort_key_val(keys, vals, descending=True)


def _merge_chunk(r1, r2, v1, v2, ck, cv):
    cr = lax.rev(ck, (0,))
    cvr = lax.rev(cv, (0,))
    m = r2 >= cr
    hi = jnp.where(m, r2, cr)
    hv = jnp.where(m, v2, cvr)
    hi, hv = _sortkv(hi, hv)
    hr = lax.rev(hi, (0,))
    hvr = lax.rev(hv, (0,))
    m2 = r1 >= hr
    ak = jnp.where(m2, r1, hr)
    av = jnp.where(m2, v1, hvr)
    bk = jnp.where(m2, hr, r1)
    bv = jnp.where(m2, hvr, v1)
    r1, v1 = _sortkv(ak, av)
    r2, v2 = _sortkv(bk, bv)
    return r1, r2, v1, v2


def _sc_body(h_hbm, at_hbm, bt_hbm, pv_hbm, ow_hbm, oi_hbm,
             hbuf, sval, sidx, w12, i12, atv, btv, pvv, wout, iout):
    wid = lax.axis_index("s") * _NC + lax.axis_index("c")
    row0 = wid * _RPW
    pltpu.sync_copy(at_hbm, atv)
    pltpu.sync_copy(bt_hbm, btv)
    pltpu.sync_copy(pv_hbm, pvv)
    iota = lax.broadcasted_iota(jnp.int32, (_L,), 0)
    negv = jnp.full((_L,), _DEAD, jnp.float32)
    bigv = jnp.full((_L,), 1 << 30, jnp.int32)

    def topk_half(hb_base):
        def fold(j, c):
            a, b = c
            va = hbuf[pl.ds(hb_base + 2 * _L * j, _L)]
            vb = hbuf[pl.ds(hb_base + 2 * _L * j + _L, _L)]
            return jnp.maximum(a, va), jnp.maximum(b, vb)

        fa, fb = lax.fori_loop(0, _PAD // (2 * _L), fold, (negv, negv))
        t = jnp.minimum(jnp.min(fa), jnp.min(fb))

        def comp(j, o):
            v = hbuf[pl.ds(hb_base + _L * j, _L)]
            msk = v >= t
            plsc.store_compressed(sval.at[pl.ds(o, _L)], v, mask=msk)
            plsc.store_compressed(sidx.at[pl.ds(o, _L)], iota + _L * j,
                                  mask=msk)
            return o + jnp.sum(msk.astype(jnp.int32))

        o = lax.fori_loop(0, _PAD // _L, comp, jnp.int32(0))
        sval[pl.ds(o, _L)] = negv
        sidx[pl.ds(o, _L)] = bigv
        sval[pl.ds(o + _L, _L)] = negv
        sidx[pl.ds(o + _L, _L)] = bigv
        nc = (o + _L - 1) // _L

        def mstep(j, c):
            ck = sval[pl.ds(_L * j, _L)]
            cv = sidx[pl.ds(_L * j, _L)]
            ck, cv = _sortkv(ck, cv)
            return _merge_chunk(*c, ck, cv)

        return lax.fori_loop(0, nc, mstep, (negv, negv, bigv, bigv))

    def row_body(r_glob):
        hb_base = (r_glob % _RB) * (2 * _PAD)
        w1a, w1b, p1a, p1b = topk_half(hb_base)
        w2a, w2b, p2a, p2b = topk_half(hb_base + _PAD)
        w12[pl.ds(0, _L)] = w1a
        w12[pl.ds(_L, _L)] = w1b
        w12[pl.ds(2 * _L, _L)] = w2a
        w12[pl.ds(3 * _L, _L)] = w2b
        i12[pl.ds(0, _L)] = p1a
        i12[pl.ds(_L, _L)] = p1b
        i12[pl.ds(2 * _L, _L)] = p2a
        i12[pl.ds(3 * _L, _L)] = p2b

        r1, r2, v1, v2 = negv, negv, bigv, bigv
        for j in range(_NCAND // _L):
            ai = atv[pl.ds(_L * j, _L)]
            bi = btv[pl.ds(_L * j, _L)]
            pv = pvv[pl.ds(_L * j, _L)]
            ga = plsc.load_gather(w12, [ai])
            gb = plsc.load_gather(w12, [bi])
            ia = plsc.load_gather(i12, [ai])
            ib = plsc.load_gather(i12, [bi])
            ck = jnp.maximum(ga + gb, 0.0) + pv
            cv = ia * _PKM + ib
            ck, cv = _sortkv(ck, cv)
            r1, r2, v1, v2 = _merge_chunk(r1, r2, v1, v2, ck, cv)

        out_off = r_glob * _K
        wout[pl.ds(out_off, _L)] = r1
        wout[pl.ds(out_off + _L, _L)] = r2
        iout[pl.ds(out_off, _L)] = v1
        iout[pl.ds(out_off + _L, _L)] = v2

    def batch_body(b, _):
        pltpu.sync_copy(
            h_hbm.at[pl.ds((row0 + b * _RB) * (2 * _PAD), _RB * 2 * _PAD)],
            hbuf)

        def rloop(r, _2):
            row_body(b * _RB + r)
            return 0

        lax.fori_loop(0, _RB, rloop, 0)
        return 0

    lax.fori_loop(0, _NBATCH, batch_body, 0)
    pltpu.sync_copy(wout, ow_hbm.at[pl.ds(row0 * _K, _RPW * _K)])
    pltpu.sync_copy(iout, oi_hbm.at[pl.ds(row0 * _K, _RPW * _K)])


def _sc_topk(h_flat, atab, btab, padv):
    mesh = plsc.VectorSubcoreMesh(core_axis_name="c", subcore_axis_name="s",
                                  num_cores=_NC, num_subcores=_NS)
    f = pl.kernel(
        _sc_body,
        out_type=(
            jax.ShapeDtypeStruct((_N_TOK * _K,), jnp.float32),
            jax.ShapeDtypeStruct((_N_TOK * _K,), jnp.int32),
        ),
        mesh=mesh,
        compiler_params=pltpu.CompilerParams(needs_layout_passes=False),
        scratch_types=[
            pltpu.VMEM((_RB * 2 * _PAD,), jnp.float32),
            pltpu.VMEM((_PAD + 2 * _L,), jnp.float32),
            pltpu.VMEM((_PAD + 2 * _L,), jnp.int32),
            pltpu.VMEM((4 * _L,), jnp.float32),
            pltpu.VMEM((4 * _L,), jnp.int32),
            pltpu.VMEM((_NCAND,), jnp.int32),
            pltpu.VMEM((_NCAND,), jnp.int32),
            pltpu.VMEM((_NCAND,), jnp.float32),
            pltpu.VMEM((_RPW * _K,), jnp.float32),
            pltpu.VMEM((_RPW * _K,), jnp.int32),
        ],
    )
    return f(h_flat, atab, btab, padv)


def _sc_tables():
    import numpy as np
    at = np.zeros((_NCAND,), np.int32)
    bt = np.zeros((_NCAND,), np.int32)
    pv = np.full((_NCAND,), _DEAD, np.float32)
    for j, (a, b) in enumerate(_AB):
        at[j] = a
        bt[j] = b + _K
        pv[j] = 0.0
    return jnp.asarray(at), jnp.asarray(bt), jnp.asarray(pv)


def kernel(x, W, b_lin, bias, k):
    del bias
    npad = _PAD - _PKM
    zrows = jnp.zeros((npad, _D_IN), W.dtype)
    wpt = jnp.concatenate([W[:_PKM], zrows, W[_PKM:], zrows], axis=0).T
    negs = jnp.full((npad,), _NEG_PAD, jnp.float32)
    bp = jnp.concatenate(
        [b_lin[:_PKM], negs, b_lin[_PKM:], negs]).reshape(1, 2 * _PAD)
    h = _matmul(x, wpt, bp)
    atab, btab, padv = _sc_tables()
    w_flat, i_flat = _sc_topk(h.reshape(-1), atab, btab, padv)
    w = w_flat.reshape(_N_TOK, _K)
    i = i_flat.reshape(_N_TOK, _K)
    keep = jnp.asarray(k) == _K
    w = jnp.where(keep, w, jnp.zeros_like(w))
    i = jnp.where(keep, i, jnp.zeros_like(i))
    return w, i


# speedup vs baseline: 134.5358x; 1.2415x over previous
import functools

import jax
import jax.numpy as jnp
from jax import lax
from jax.experimental import pallas as pl
from jax.experimental.pallas import tpu as pltpu
from jax.experimental.pallas import tpu_sc as plsc

_D_IN = 2048
_PKM = 1000
_PAD = 1024
_K = 32
_N_TOK = 8192
_BLK = 256
_NEG_PAD = -1e30
_DEAD = -3e38

_NC = 2
_NS = 16
_L = 16
_NW = _NC * _NS
_RPW = _N_TOK // _NW
_RB = 16
_NBATCH = _RPW // _RB

_AB = [(a, b) for a in range(_K) for b in range(_K) if (a + 1) * (b + 1) <= _K]
_NCAND = 128


def _mm_body(x_ref, w_ref, b_ref, h_ref, t_ref):
    h = jnp.dot(x_ref[...], w_ref[...], preferred_element_type=jnp.float32)
    h = h + b_ref[...]
    h_ref[...] = h

    def thresh(v):
        w = _PAD
        while w > _K:
            w //= 2
            v = jnp.maximum(v[:, :w], v[:, w:])
        return jnp.min(v, axis=1, keepdims=True)

    t_ref[...] = jnp.concatenate(
        [thresh(h[:, :_PAD]), thresh(h[:, _PAD:])], axis=1)


def _matmul(x, wpt, bp):
    fixed = lambda i: (0, 0)
    return pl.pallas_call(
        _mm_body,
        grid=(_N_TOK // _BLK,),
        in_specs=[
            pl.BlockSpec((_BLK, _D_IN), lambda i: (i, 0)),
            pl.BlockSpec((_D_IN, 2 * _PAD), fixed),
            pl.BlockSpec((1, 2 * _PAD), fixed),
        ],
        out_specs=[
            pl.BlockSpec((_BLK, 2 * _PAD), lambda i: (i, 0)),
            pl.BlockSpec((_BLK, 2), lambda i: (i, 0)),
        ],
        out_shape=[
            jax.ShapeDtypeStruct((_N_TOK, 2 * _PAD), jnp.float32),
            jax.ShapeDtypeStruct((_N_TOK, 2), jnp.float32),
        ],
        compiler_params=pltpu.CompilerParams(
            dimension_semantics=("parallel",),
        ),
    )(x, wpt, bp)


def _sortkv(keys, vals):
    return plsc.sort_key_val(keys, vals, descending=True)


def _minmax_kv(ak, av, bk, bv):
    m = ak >= bk
    hk = jnp.where(m, ak, bk)
    hv = jnp.where(m, av, bv)
    lk = jnp.where(m, bk, ak)
    lv = jnp.where(m, bv, av)
    return hk, hv, lk, lv


def _merge16_full(c1k, c1v, c2k, c2v):
    r2k = lax.rev(c2k, (0,))
    r2v = lax.rev(c2v, (0,))
    hk, hv, lk, lv = _minmax_kv(c1k, c1v, r2k, r2v)
    t1k, t1v = _sortkv(hk, hv)
    t2k, t2v = _sortkv(lk, lv)
    return t1k, t1v, t2k, t2v


def _merge32_top(r1k, r1v, r2k, r2v, t1k, t1v, t2k, t2v):
    x1k, x1v, _, _ = _minmax_kv(r1k, r1v, lax.rev(t2k, (0,)),
                                lax.rev(t2v, (0,)))
    x2k, x2v, _, _ = _minmax_kv(r2k, r2v, lax.rev(t1k, (0,)),
                                lax.rev(t1v, (0,)))
    y1k, y1v, y2k, y2v = _minmax_kv(x1k, x1v, x2k, x2v)
    r1k, r1v = _sortkv(y1k, y1v)
    r2k, r2v = _sortkv(y2k, y2v)
    return r1k, r2k, r1v, r2v


def _sc_body(h_hbm, t_hbm, at_hbm, bt_hbm, pv_hbm, ow_hbm, oi_hbm,
             hbuf, tbuf, sval, sidx, w12, i12, atv, btv, pvv, wout, iout):
    wid = lax.axis_index("s") * _NC + lax.axis_index("c")
    row0 = wid * _RPW
    pltpu.sync_copy(at_hbm, atv)
    pltpu.sync_copy(bt_hbm, btv)
    pltpu.sync_copy(pv_hbm, pvv)
    iota = lax.broadcasted_iota(jnp.int32, (_L,), 0)
    negv = jnp.full((_L,), _DEAD, jnp.float32)
    bigv = jnp.full((_L,), 1 << 30, jnp.int32)

    def topk_half(hb_base, t):
        def comp(j, o):
            base = hb_base + 2 * _L * j
            v1 = hbuf[pl.ds(base, _L)]
            v2 = hbuf[pl.ds(base + _L, _L)]
            m1 = v1 >= t
            m2 = v2 >= t
            plsc.store_compressed(sval.at[pl.ds(o, _L)], v1, mask=m1)
            plsc.store_compressed(sidx.at[pl.ds(o, _L)],
                                  iota + 2 * _L * j, mask=m1)
            o1 = o + plsc.all_reduce_population_count(m1)[0]
            plsc.store_compressed(sval.at[pl.ds(o1, _L)], v2, mask=m2)
            plsc.store_compressed(sidx.at[pl.ds(o1, _L)],
                                  iota + 2 * _L * j + _L, mask=m2)
            return o1 + plsc.all_reduce_population_count(m2)[0]

        o = lax.fori_loop(0, _PAD // (2 * _L), comp, jnp.int32(0))
        sval[pl.ds(o, _L)] = negv
        sidx[pl.ds(o, _L)] = bigv
        sval[pl.ds(o + _L, _L)] = negv
        sidx[pl.ds(o + _L, _L)] = bigv
        npair = (o + 2 * _L - 1) // (2 * _L)

        def mstep(j, c):
            r1k, r2k, r1v, r2v = c
            b = 2 * _L * j
            c1k, c1v = _sortkv(sval[pl.ds(b, _L)], sidx[pl.ds(b, _L)])
            c2k, c2v = _sortkv(sval[pl.ds(b + _L, _L)],
                               sidx[pl.ds(b + _L, _L)])
            t1k, t1v, t2k, t2v = _merge16_full(c1k, c1v, c2k, c2v)
            return _merge32_top(r1k, r1v, r2k, r2v, t1k, t1v, t2k, t2v)

        r1k, r2k, r1v, r2v = lax.fori_loop(0, npair, mstep,
                                           (negv, negv, bigv, bigv))
        return r1k, r2k, r1v, r2v

    def row_body(r_glob):
        rl = r_glob % _RB
        hb_base = rl * (2 * _PAD)
        tv = tbuf[pl.ds(2 * rl, _L)]
        w1a, w1b, p1a, p1b = topk_half(hb_base, tv[0])
        w2a, w2b, p2a, p2b = topk_half(hb_base + _PAD, tv[1])
        w12[pl.ds(0, _L)] = w1a
        w12[pl.ds(_L, _L)] = w1b
        w12[pl.ds(2 * _L, _L)] = w2a
        w12[pl.ds(3 * _L, _L)] = w2b
        i12[pl.ds(0, _L)] = p1a
        i12[pl.ds(_L, _L)] = p1b
        i12[pl.ds(2 * _L, _L)] = p2a
        i12[pl.ds(3 * _L, _L)] = p2b

        def cchunk(jj):
            ai = atv[pl.ds(_L * jj, _L)]
            bi = btv[pl.ds(_L * jj, _L)]
            pv = pvv[pl.ds(_L * jj, _L)]
            ga = plsc.load_gather(w12, [ai])
            gb = plsc.load_gather(w12, [bi])
            ia = plsc.load_gather(i12, [ai])
            ib = plsc.load_gather(i12, [bi])
            ck = jnp.maximum(ga + gb, 0.0) + pv
            cv = ia * _PKM + ib
            return _sortkv(ck, cv)

        r1, r2, v1, v2 = negv, negv, bigv, bigv
        for j in range(_NCAND // (2 * _L)):
            c1k, c1v = cchunk(2 * j)
            c2k, c2v = cchunk(2 * j + 1)
            t1k, t1v, t2k, t2v = _merge16_full(c1k, c1v, c2k, c2v)
            r1, r2, v1, v2 = _merge32_top(r1, v1, r2, v2,
                                          t1k, t1v, t2k, t2v)

        out_off = r_glob * _K
        wout[pl.ds(out_off, _L)] = r1
        wout[pl.ds(out_off + _L, _L)] = r2
        iout[pl.ds(out_off, _L)] = v1
        iout[pl.ds(out_off + _L, _L)] = v2

    def batch_body(b, _):
        pltpu.sync_copy(
            h_hbm.at[pl.ds((row0 + b * _RB) * (2 * _PAD), _RB * 2 * _PAD)],
            hbuf)
        pltpu.sync_copy(
            t_hbm.at[pl.ds((row0 + b * _RB) * 2, _RB * 2)],
            tbuf.at[pl.ds(0, _RB * 2)])

        def rloop(r, _2):
            row_body(b * _RB + r)
            return 0

        lax.fori_loop(0, _RB, rloop, 0)
        return 0

    lax.fori_loop(0, _NBATCH, batch_body, 0)
    pltpu.sync_copy(wout, ow_hbm.at[pl.ds(row0 * _K, _RPW * _K)])
    pltpu.sync_copy(iout, oi_hbm.at[pl.ds(row0 * _K, _RPW * _K)])


def _sc_topk(h_flat, t_flat, atab, btab, padv):
    mesh = plsc.VectorSubcoreMesh(core_axis_name="c", subcore_axis_name="s",
                                  num_cores=_NC, num_subcores=_NS)
    f = pl.kernel(
        _sc_body,
        out_type=(
            jax.ShapeDtypeStruct((_N_TOK * _K,), jnp.float32),
            jax.ShapeDtypeStruct((_N_TOK * _K,), jnp.int32),
        ),
        mesh=mesh,
        compiler_params=pltpu.CompilerParams(needs_layout_passes=False),
        scratch_types=[
            pltpu.VMEM((_RB * 2 * _PAD,), jnp.float32),
            pltpu.VMEM((_RB * 2 + _L,), jnp.float32),
            pltpu.VMEM((_PAD + 2 * _L,), jnp.float32),
            pltpu.VMEM((_PAD + 2 * _L,), jnp.int32),
            pltpu.VMEM((4 * _L,), jnp.float32),
            pltpu.VMEM((4 * _L,), jnp.int32),
            pltpu.VMEM((_NCAND,), jnp.int32),
            pltpu.VMEM((_NCAND,), jnp.int32),
            pltpu.VMEM((_NCAND,), jnp.float32),
            pltpu.VMEM((_RPW * _K,), jnp.float32),
            pltpu.VMEM((_RPW * _K,), jnp.int32),
        ],
    )
    return f(h_flat, t_flat, atab, btab, padv)


def _sc_tables():
    import numpy as np
    at = np.zeros((_NCAND,), np.int32)
    bt = np.zeros((_NCAND,), np.int32)
    pv = np.full((_NCAND,), _DEAD, np.float32)
    for j, (a, b) in enumerate(_AB):
        at[j] = a
        bt[j] = b + _K
        pv[j] = 0.0
    return jnp.asarray(at), jnp.asarray(bt), jnp.asarray(pv)


def kernel(x, W, b_lin, bias, k):
    del bias
    npad = _PAD - _PKM
    zrows = jnp.zeros((npad, _D_IN), W.dtype)
    wpt = jnp.concatenate([W[:_PKM], zrows, W[_PKM:], zrows], axis=0).T
    negs = jnp.full((npad,), _NEG_PAD, jnp.float32)
    bp = jnp.concatenate(
        [b_lin[:_PKM], negs, b_lin[_PKM:], negs]).reshape(1, 2 * _PAD)
    h, tt = _matmul(x, wpt, bp)
    atab, btab, padv = _sc_tables()
    w_flat, i_flat = _sc_topk(h.reshape(-1), tt.reshape(-1), atab, btab, padv)
    w = w_flat.reshape(_N_TOK, _K)
    i = i_flat.reshape(_N_TOK, _K)
    keep = jnp.asarray(k) == _K
    w = jnp.where(keep, w, jnp.zeros_like(w))
    i = jnp.where(keep, i, jnp.zeros_like(i))
    return w, i


# speedup vs baseline: 164.7242x; 1.2244x over previous
import functools

import jax
import jax.numpy as jnp
from jax import lax
from jax.experimental import pallas as pl
from jax.experimental.pallas import tpu as pltpu
from jax.experimental.pallas import tpu_sc as plsc

_D_IN = 2048
_PKM = 1000
_PAD = 1024
_K = 32
_N_TOK = 8192
_BLK = 256
_NEG_PAD = -1e30
_DEAD = -3e38

_NC = 2
_NS = 16
_L = 16
_NW = _NC * _NS
_RPW = _N_TOK // _NW
_RB = 16
_NBATCH = _RPW // _RB

_AB = [(a, b) for a in range(_K) for b in range(_K) if (a + 1) * (b + 1) <= _K]
_NCAND = 128


def _mm_body(x_ref, w_ref, b_ref, h_ref, t_ref):
    h = jnp.dot(x_ref[...], w_ref[...], preferred_element_type=jnp.float32)
    h = h + b_ref[...]
    h_ref[...] = h

    def thresh(v):
        w = _PAD
        while w > _K:
            w //= 2
            v = jnp.maximum(v[:, :w], v[:, w:])
        return jnp.min(v, axis=1, keepdims=True)

    t_ref[...] = jnp.concatenate(
        [thresh(h[:, :_PAD]), thresh(h[:, _PAD:])], axis=1)


def _matmul(x, wpt, bp):
    fixed = lambda i: (0, 0)
    return pl.pallas_call(
        _mm_body,
        grid=(_N_TOK // _BLK,),
        in_specs=[
            pl.BlockSpec((_BLK, _D_IN), lambda i: (i, 0)),
            pl.BlockSpec((_D_IN, 2 * _PAD), fixed),
            pl.BlockSpec((1, 2 * _PAD), fixed),
        ],
        out_specs=[
            pl.BlockSpec((_BLK, 2 * _PAD), lambda i: (i, 0)),
            pl.BlockSpec((_BLK, 2), lambda i: (i, 0)),
        ],
        out_shape=[
            jax.ShapeDtypeStruct((_N_TOK, 2 * _PAD), jnp.float32),
            jax.ShapeDtypeStruct((_N_TOK, 2), jnp.float32),
        ],
        compiler_params=pltpu.CompilerParams(
            dimension_semantics=("parallel",),
        ),
    )(x, wpt, bp)


def _sortkv(keys, vals):
    return plsc.sort_key_val(keys, vals, descending=True)


def _minmax_kv(ak, av, bk, bv):
    m = ak >= bk
    hk = jnp.where(m, ak, bk)
    hv = jnp.where(m, av, bv)
    lk = jnp.where(m, bk, ak)
    lv = jnp.where(m, bv, av)
    return hk, hv, lk, lv


def _merge16_full(c1k, c1v, c2k, c2v):
    r2k = lax.rev(c2k, (0,))
    r2v = lax.rev(c2v, (0,))
    hk, hv, lk, lv = _minmax_kv(c1k, c1v, r2k, r2v)
    t1k, t1v = _sortkv(hk, hv)
    t2k, t2v = _sortkv(lk, lv)
    return t1k, t1v, t2k, t2v


def _merge32_top(r1k, r1v, r2k, r2v, t1k, t1v, t2k, t2v):
    x1k, x1v, _, _ = _minmax_kv(r1k, r1v, lax.rev(t2k, (0,)),
                                lax.rev(t2v, (0,)))
    x2k, x2v, _, _ = _minmax_kv(r2k, r2v, lax.rev(t1k, (0,)),
                                lax.rev(t1v, (0,)))
    y1k, y1v, y2k, y2v = _minmax_kv(x1k, x1v, x2k, x2v)
    r1k, r1v = _sortkv(y1k, y1v)
    r2k, r2v = _sortkv(y2k, y2v)
    return r1k, r2k, r1v, r2v


def _sc_body(h_hbm, t_hbm, at_hbm, bt_hbm, pv_hbm, ow_hbm, oi_hbm,
             hbuf, tbuf, sval, sidx, sval2, sidx2, w12, i12, atv, btv, pvv,
             wout, iout):
    wid = lax.axis_index("s") * _NC + lax.axis_index("c")
    row0 = wid * _RPW
    pltpu.sync_copy(at_hbm, atv)
    pltpu.sync_copy(bt_hbm, btv)
    pltpu.sync_copy(pv_hbm, pvv)
    iota = lax.broadcasted_iota(jnp.int32, (_L,), 0)
    negv = jnp.full((_L,), _DEAD, jnp.float32)
    bigv = jnp.full((_L,), 1 << 30, jnp.int32)

    def compact_both(hb_base, t1, t2):
        def comp(j, c):
            oa, ob = c
            basea = hb_base + 2 * _L * j
            baseb = basea + _PAD
            va1 = hbuf[pl.ds(basea, _L)]
            va2 = hbuf[pl.ds(basea + _L, _L)]
            vb1 = hbuf[pl.ds(baseb, _L)]
            vb2 = hbuf[pl.ds(baseb + _L, _L)]
            ma1 = va1 >= t1
            ma2 = va2 >= t1
            mb1 = vb1 >= t2
            mb2 = vb2 >= t2
            ix = iota + 2 * _L * j
            plsc.store_compressed(sval.at[pl.ds(oa, _L)], va1, mask=ma1)
            plsc.store_compressed(sidx.at[pl.ds(oa, _L)], ix, mask=ma1)
            plsc.store_compressed(sval2.at[pl.ds(ob, _L)], vb1, mask=mb1)
            plsc.store_compressed(sidx2.at[pl.ds(ob, _L)], ix, mask=mb1)
            oa1 = oa + plsc.all_reduce_population_count(ma1)[0]
            ob1 = ob + plsc.all_reduce_population_count(mb1)[0]
            plsc.store_compressed(sval.at[pl.ds(oa1, _L)], va2, mask=ma2)
            plsc.store_compressed(sidx.at[pl.ds(oa1, _L)], ix + _L, mask=ma2)
            plsc.store_compressed(sval2.at[pl.ds(ob1, _L)], vb2, mask=mb2)
            plsc.store_compressed(sidx2.at[pl.ds(ob1, _L)], ix + _L,
                                  mask=mb2)
            return (oa1 + plsc.all_reduce_population_count(ma2)[0],
                    ob1 + plsc.all_reduce_population_count(mb2)[0])

        oa, ob = lax.fori_loop(0, _PAD // (2 * _L), comp,
                               (jnp.int32(0), jnp.int32(0)))
        sval[pl.ds(oa, _L)] = negv
        sidx[pl.ds(oa, _L)] = bigv
        sval[pl.ds(oa + _L, _L)] = negv
        sidx[pl.ds(oa + _L, _L)] = bigv
        sval2[pl.ds(ob, _L)] = negv
        sidx2[pl.ds(ob, _L)] = bigv
        sval2[pl.ds(ob + _L, _L)] = negv
        sidx2[pl.ds(ob + _L, _L)] = bigv
        return oa, ob

    def merge_surv(vref, iref, o):
        npair = (o + 2 * _L - 1) // (2 * _L)

        def mstep(j, c):
            r1k, r2k, r1v, r2v = c
            b = 2 * _L * j
            c1k, c1v = _sortkv(vref[pl.ds(b, _L)], iref[pl.ds(b, _L)])
            c2k, c2v = _sortkv(vref[pl.ds(b + _L, _L)],
                               iref[pl.ds(b + _L, _L)])
            t1k, t1v, t2k, t2v = _merge16_full(c1k, c1v, c2k, c2v)
            return _merge32_top(r1k, r1v, r2k, r2v, t1k, t1v, t2k, t2v)

        return lax.fori_loop(0, npair, mstep, (negv, negv, bigv, bigv))

    def row_body(r_glob):
        rl = r_glob % _RB
        hb_base = rl * (2 * _PAD)
        tv = tbuf[pl.ds(2 * rl, _L)]
        oa, ob = compact_both(hb_base, tv[0], tv[1])
        w1a, w1b, p1a, p1b = merge_surv(sval, sidx, oa)
        w2a, w2b, p2a, p2b = merge_surv(sval2, sidx2, ob)
        w12[pl.ds(0, _L)] = w1a
        w12[pl.ds(_L, _L)] = w1b
        w12[pl.ds(2 * _L, _L)] = w2a
        w12[pl.ds(3 * _L, _L)] = w2b
        i12[pl.ds(0, _L)] = p1a
        i12[pl.ds(_L, _L)] = p1b
        i12[pl.ds(2 * _L, _L)] = p2a
        i12[pl.ds(3 * _L, _L)] = p2b

        def cchunk(jj):
            ai = atv[pl.ds(_L * jj, _L)]
            bi = btv[pl.ds(_L * jj, _L)]
            pv = pvv[pl.ds(_L * jj, _L)]
            ga = plsc.load_gather(w12, [ai])
            gb = plsc.load_gather(w12, [bi])
            ia = plsc.load_gather(i12, [ai])
            ib = plsc.load_gather(i12, [bi])
            ck = jnp.maximum(ga + gb, 0.0) + pv
            cv = ia * _PKM + ib
            return _sortkv(ck, cv)

        r1, r2, v1, v2 = negv, negv, bigv, bigv
        for j in range(_NCAND // (2 * _L)):
            c1k, c1v = cchunk(2 * j)
            c2k, c2v = cchunk(2 * j + 1)
            t1k, t1v, t2k, t2v = _merge16_full(c1k, c1v, c2k, c2v)
            r1, r2, v1, v2 = _merge32_top(r1, v1, r2, v2,
                                          t1k, t1v, t2k, t2v)

        out_off = r_glob * _K
        wout[pl.ds(out_off, _L)] = r1
        wout[pl.ds(out_off + _L, _L)] = r2
        iout[pl.ds(out_off, _L)] = v1
        iout[pl.ds(out_off + _L, _L)] = v2

    def batch_body(b, _):
        pltpu.sync_copy(
            h_hbm.at[pl.ds((row0 + b * _RB) * (2 * _PAD), _RB * 2 * _PAD)],
            hbuf)
        pltpu.sync_copy(
            t_hbm.at[pl.ds((row0 + b * _RB) * 2, _RB * 2)],
            tbuf.at[pl.ds(0, _RB * 2)])

        def rloop(r, _2):
            row_body(b * _RB + r)
            return 0

        lax.fori_loop(0, _RB, rloop, 0)
        return 0

    lax.fori_loop(0, _NBATCH, batch_body, 0)
    pltpu.sync_copy(wout, ow_hbm.at[pl.ds(row0 * _K, _RPW * _K)])
    pltpu.sync_copy(iout, oi_hbm.at[pl.ds(row0 * _K, _RPW * _K)])


def _sc_topk(h_flat, t_flat, atab, btab, padv):
    mesh = plsc.VectorSubcoreMesh(core_axis_name="c", subcore_axis_name="s",
                                  num_cores=_NC, num_subcores=_NS)
    f = pl.kernel(
        _sc_body,
        out_type=(
            jax.ShapeDtypeStruct((_N_TOK * _K,), jnp.float32),
            jax.ShapeDtypeStruct((_N_TOK * _K,), jnp.int32),
        ),
        mesh=mesh,
        compiler_params=pltpu.CompilerParams(needs_layout_passes=False),
        scratch_types=[
            pltpu.VMEM((_RB * 2 * _PAD,), jnp.float32),
            pltpu.VMEM((_RB * 2 + _L,), jnp.float32),
            pltpu.VMEM((_PAD + 2 * _L,), jnp.float32),
            pltpu.VMEM((_PAD + 2 * _L,), jnp.int32),
            pltpu.VMEM((_PAD + 2 * _L,), jnp.float32),
            pltpu.VMEM((_PAD + 2 * _L,), jnp.int32),
            pltpu.VMEM((4 * _L,), jnp.float32),
            pltpu.VMEM((4 * _L,), jnp.int32),
            pltpu.VMEM((_NCAND,), jnp.int32),
            pltpu.VMEM((_NCAND,), jnp.int32),
            pltpu.VMEM((_NCAND,), jnp.float32),
            pltpu.VMEM((_RPW * _K,), jnp.float32),
            pltpu.VMEM((_RPW * _K,), jnp.int32),
        ],
    )
    return f(h_flat, t_flat, atab, btab, padv)


def _sc_tables():
    import numpy as np
    at = np.zeros((_NCAND,), np.int32)
    bt = np.zeros((_NCAND,), np.int32)
    pv = np.full((_NCAND,), _DEAD, np.float32)
    for j, (a, b) in enumerate(_AB):
        at[j] = a
        bt[j] = b + _K
        pv[j] = 0.0
    return jnp.asarray(at), jnp.asarray(bt), jnp.asarray(pv)


def kernel(x, W, b_lin, bias, k):
    del bias
    npad = _PAD - _PKM
    zrows = jnp.zeros((npad, _D_IN), W.dtype)
    wpt = jnp.concatenate([W[:_PKM], zrows, W[_PKM:], zrows], axis=0).T
    negs = jnp.full((npad,), _NEG_PAD, jnp.float32)
    bp = jnp.concatenate(
        [b_lin[:_PKM], negs, b_lin[_PKM:], negs]).reshape(1, 2 * _PAD)
    h, tt = _matmul(x, wpt, bp)
    atab, btab, padv = _sc_tables()
    w_flat, i_flat = _sc_topk(h.reshape(-1), tt.reshape(-1), atab, btab, padv)
    w = w_flat.reshape(_N_TOK, _K)
    i = i_flat.reshape(_N_TOK, _K)
    keep = jnp.asarray(k) == _K
    w = jnp.where(keep, w, jnp.zeros_like(w))
    i = jnp.where(keep, i, jnp.zeros_like(i))
    return w, i
